# baseline (speedup 1.0000x reference)
import jax
import jax.numpy as jnp
from jax import lax
from jax.experimental import pallas as pl
from jax.experimental.pallas import tpu as pltpu
from jax.experimental.pallas import tpu_sc as plsc

N, E = 10000, 320000
ND, ED, NB = 128, 128, 20
HID = ND + 2 * ED
PW = 512
NPAD = 10240
RANGE = 640
ROUNDS = 8
ACC_ROWS = RANGE + 8
DUMMY = RANGE
NC, NS, L = 2, 16, 16
EPT = E // NS
SEG = 800
NSEG = EPT // SEG
CCAP = 832
TRASH = SEG
K = 32
MAXCH = CCAP // K
ROWS_PT = RANGE // NS


def _mlp_body(x_ref, w1_ref, b1_ref, w2_ref, b2_ref, o_ref):
    h = jnp.dot(x_ref[...], w1_ref[...], preferred_element_type=jnp.float32)
    h = h + b1_ref[...]
    h = h * jax.nn.sigmoid(h)
    o_ref[...] = jnp.dot(h, w2_ref[...], preferred_element_type=jnp.float32) + b2_ref[...]


def _mlp(x, W1, b1, W2, b2):
    blk = 512
    return pl.pallas_call(
        _mlp_body,
        grid=(NPAD // blk,),
        in_specs=[
            pl.BlockSpec((blk, ND), lambda i: (i, 0)),
            pl.BlockSpec((ND, ND), lambda i: (0, 0)),
            pl.BlockSpec((1, ND), lambda i: (0, 0)),
            pl.BlockSpec((ND, HID), lambda i: (0, 0)),
            pl.BlockSpec((1, HID), lambda i: (0, 0)),
        ],
        out_specs=pl.BlockSpec((blk, HID), lambda i: (i, 0)),
        out_shape=jax.ShapeDtypeStruct((NPAD, HID), jnp.float32),
    )(x, W1, b1.reshape(1, ND), W2, b2.reshape(1, HID))


def _edge_pack_body(rbf_ref, env_ref, rsh_ref, wr_ref, br_ref, o_ref):
    t = jnp.dot(rbf_ref[...], wr_ref[...], preferred_element_type=jnp.float32)
    fw = (t + br_ref[...]) * env_ref[...]
    rshp = jnp.pad(rsh_ref[...], ((0, 0), (0, PW - HID - 4)))
    o_ref[...] = jnp.concatenate([fw, rshp], axis=1)


def _edge_pack(rbf, envelope, rsh4, Wr, br):
    blk = 1280
    return pl.pallas_call(
        _edge_pack_body,
        grid=(E // blk,),
        in_specs=[
            pl.BlockSpec((blk, NB), lambda i: (i, 0)),
            pl.BlockSpec((blk, 1), lambda i: (i, 0)),
            pl.BlockSpec((blk, 4), lambda i: (i, 0)),
            pl.BlockSpec((NB, HID), lambda i: (0, 0)),
            pl.BlockSpec((1, HID), lambda i: (0, 0)),
        ],
        out_specs=pl.BlockSpec((blk, PW), lambda i: (i, 0)),
        out_shape=jax.ShapeDtypeStruct((E, PW), jnp.float32),
    )(rbf, envelope, rsh4, Wr, br.reshape(1, HID))


def _sc_body(t_hbm, p_hbm, src_hbm, dst_hbm,
             x0_hbm, x1_hbm, x2_hbm, x3_hbm,
             o0_hbm, o1_hbm, o2_hbm, o3_hbm,
             acc0, acc1, acc2, acc3, ebuf, sbuf,
             dseg, sseg, posb, pst, dummy, ecomp, scomp, didx2,
             tA, tB, gPA, gPB, msg,
             gsemA, gsemB, ssem):
    c = lax.axis_index("c")
    s = lax.axis_index("s")
    lanes = lax.iota(jnp.int32, L)
    sh_idx = [jnp.maximum(lanes - sh, 0) for sh in (1, 2, 4, 8)]
    sh_msk = [lanes >= sh for sh in (1, 2, 4, 8)]
    region = s * CCAP
    accs = (acc0, acc1, acc2, acc3)
    xs = (x0_hbm, x1_hbm, x2_hbm, x3_hbm)
    os_ = (o0_hbm, o1_hbm, o2_hbm, o3_hbm)
    tb = (tA, tB)
    gP = (gPA, gPB)
    gsems = (gsemA, gsemB)

    def fill(i, _):
        dummy[pl.ds(i * L, L)] = jnp.zeros((L,), jnp.int32) + (DUMMY << 10)
        return 0
    lax.fori_loop(0, CCAP // L, fill, 0)

    def per_round(r, _):
        base = (2 * r + c) * RANGE

        row0 = s * ROWS_PT
        for q in range(4):
            pltpu.sync_copy(xs[q].at[pl.ds(base + row0, ROWS_PT)],
                            accs[q].at[pl.ds(row0, ROWS_PT)])
        @pl.when(s == 0)
        def _():
            for q in range(4):
                pltpu.sync_copy(xs[q].at[pl.ds(0, 8)],
                                accs[q].at[pl.ds(RANGE, 8)])
        plsc.subcore_barrier()

        def per_seg(g, _):
            row = s * NSEG + g
            e0 = row * SEG
            pltpu.sync_copy(dst_hbm.at[row], dseg)
            pltpu.sync_copy(src_hbm.at[row], sseg)

            def scan(i, cnt):
                d = dseg[pl.ds(i * L, L)]
                dl = d - base
                m = (dl >= 0) & (dl < RANGE)
                v = jnp.where(m, 1, 0)
                for ix, mk in zip(sh_idx, sh_msk):
                    g2 = v.at[ix].get(mode="promise_in_bounds")
                    v = v + jnp.where(mk, g2, 0)
                posb[pl.ds(i * L, L)] = jnp.where(m, cnt + v - 1, TRASH) + region
                pst[pl.ds(i * L, L)] = (
                    (jnp.where(m, dl, DUMMY) << 10) + i * L + lanes)
                return cnt + v[L - 1]

            cnt = lax.fori_loop(0, SEG // L, scan, jnp.int32(0))

            pltpu.sync_copy(dummy, ebuf.at[pl.ds(region, CCAP)])
            pltpu.sync_copy(pst, ebuf.at[posb])
            pltpu.sync_copy(sseg, sbuf.at[posb])

            pltpu.sync_copy(ebuf.at[pl.ds(region, CCAP)], ecomp)
            pltpu.sync_copy(sbuf.at[pl.ds(region, CCAP)], scomp)

            def shift(i, _):
                pk = ecomp[pl.ds(i * L, L)]
                dl = jnp.minimum(jnp.maximum(pk >> 10, 0), DUMMY)
                didx2[i >> 1, pl.ds((i & 1) * L, L)] = dl
                ecomp[pl.ds(i * L, L)] = (pk & 1023) + e0
                sv = scomp[pl.ds(i * L, L)]
                scomp[pl.ds(i * L, L)] = jnp.minimum(jnp.maximum(sv, 0), N - 1)
                return 0
            lax.fori_loop(0, CCAP // L, shift, 0)

            nchunks = (cnt + K - 1) // K

            def step(j, _):
                for p in (0, 1):
                    q = 1 - p

                    @pl.when(j % 2 == p)
                    def _():
                        @pl.when(j < nchunks)
                        def _():
                            off = j * K
                            pltpu.async_copy(
                                t_hbm.at[scomp.at[pl.ds(off, K)]], tb[p], gsems[p])
                            pltpu.async_copy(
                                p_hbm.at[ecomp.at[pl.ds(off, K)]], gP[p], gsems[p])

                        @pl.when(j >= 1)
                        def _():
                            pltpu.make_async_copy(
                                t_hbm.at[pl.ds(0, K)], tb[q], gsems[q]).wait()
                            pltpu.make_async_copy(
                                p_hbm.at[pl.ds(0, K)], gP[q], gsems[q]).wait()

                            @pl.when(j >= 2)
                            def _():
                                pltpu.make_async_copy(
                                    x0_hbm.at[pl.ds(0, 4 * K)], msg, ssem).wait()

                            def edge(k, _):
                                rv = gP[q][k, pl.ds(HID, L)]
                                for grp in range(ND // L):
                                    lo = grp * L
                                    msg[k, pl.ds(lo, L)] = (
                                        tb[q][k, pl.ds(lo, L)] * gP[q][k, pl.ds(lo, L)])
                                    gev = tb[q][k, pl.ds(ND + lo, L)] * gP[q][k, pl.ds(ND + lo, L)]
                                    gsv = tb[q][k, pl.ds(2 * ND + lo, L)] * gP[q][k, pl.ds(2 * ND + lo, L)]
                                    msg[K + k, pl.ds(lo, L)] = (
                                        tb[q][k, pl.ds(HID + lo, L)] * gsv + gev * rv[0])
                                    msg[2 * K + k, pl.ds(lo, L)] = (
                                        tb[q][k, pl.ds(HID + ED + lo, L)] * gsv + gev * rv[1])
                                    msg[3 * K + k, pl.ds(lo, L)] = (
                                        tb[q][k, pl.ds(HID + 2 * ED + lo, L)] * gsv + gev * rv[2])
                                return 0

                            lax.fori_loop(0, K, edge, 0)
                            for i in range(4):
                                pltpu.async_copy(
                                    msg.at[pl.ds(i * K, K)],
                                    accs[i].at[didx2.at[j - 1]], ssem, add=True)
                return 0

            lax.fori_loop(0, nchunks + 1, step, 0)

            @pl.when(nchunks >= 1)
            def _():
                pltpu.make_async_copy(
                    x0_hbm.at[pl.ds(0, 4 * K)], msg, ssem).wait()
            return 0

        lax.fori_loop(0, NSEG, per_seg, 0)
        plsc.subcore_barrier()

        for q in range(4):
            pltpu.sync_copy(accs[q].at[pl.ds(row0, ROWS_PT)],
                            os_[q].at[pl.ds(base + row0, ROWS_PT)])
        plsc.subcore_barrier()
        return 0

    lax.fori_loop(0, ROUNDS, per_round, 0)


def _sc_message(t, p, src, dst, xq):
    mesh = plsc.VectorSubcoreMesh(core_axis_name="c", subcore_axis_name="s",
                                  num_cores=NC, num_subcores=NS)
    f32, i32 = jnp.float32, jnp.int32
    out_t = jax.ShapeDtypeStruct((NPAD, ND), f32)
    kfn = pl.kernel(
        _sc_body,
        out_type=(out_t, out_t, out_t, out_t),
        mesh=mesh,
        scratch_types=[
            pltpu.VMEM_SHARED((ACC_ROWS, ND), f32),
            pltpu.VMEM_SHARED((ACC_ROWS, ND), f32),
            pltpu.VMEM_SHARED((ACC_ROWS, ND), f32),
            pltpu.VMEM_SHARED((ACC_ROWS, ND), f32),
            pltpu.VMEM_SHARED((NS * CCAP,), i32),
            pltpu.VMEM_SHARED((NS * CCAP,), i32),
            pltpu.VMEM((SEG,), i32),
            pltpu.VMEM((SEG,), i32),
            pltpu.VMEM((SEG,), i32),
            pltpu.VMEM((SEG,), i32),
            pltpu.VMEM((CCAP,), i32),
            pltpu.VMEM((CCAP,), i32),
            pltpu.VMEM((CCAP,), i32),
            pltpu.VMEM((MAXCH, K), i32),
            pltpu.VMEM((K, 2 * HID), f32),
            pltpu.VMEM((K, 2 * HID), f32),
            pltpu.VMEM((K, PW), f32),
            pltpu.VMEM((K, PW), f32),
            pltpu.VMEM((4 * K, ND), f32),
            pltpu.SemaphoreType.DMA,
            pltpu.SemaphoreType.DMA,
            pltpu.SemaphoreType.DMA,
        ],
    )
    return kfn(t, p, src, dst, *xq)


def kernel(x_scalar, x_vector, rbf, envelope, rsh, edge_index, W1, b1, W2, b2, Wr, br):
    xs_pad = jnp.pad(x_scalar, ((0, NPAD - N), (0, 0)))
    sout = _mlp(xs_pad, W1, b1, W2, b2)
    rsh4 = jnp.pad(rsh, ((0, 0), (0, 1)))
    p = _edge_pack(rbf, envelope, rsh4, Wr, br)

    xvec = x_vector.reshape(N, 3 * ED)
    t = jnp.concatenate(
        [sout, jnp.pad(xvec, ((0, NPAD - N), (0, 0)))], axis=1)
    src = edge_index[1].astype(jnp.int32).reshape(E // SEG, SEG)
    dst = edge_index[0].astype(jnp.int32).reshape(E // SEG, SEG)
    pad_n = ((0, NPAD - N), (0, 0))
    xq = [xs_pad] + [jnp.pad(x_vector[:, q, :], pad_n) for q in range(3)]

    o0, o1, o2, o3 = _sc_message(t, p, src, dst, xq)
    new_scalar = o0[:N]
    new_vector = jnp.stack([o1[:N], o2[:N], o3[:N]], axis=1)
    return new_scalar, new_vector


# speedup vs baseline: 1.2540x; 1.2540x over previous
import jax
import jax.numpy as jnp
from jax import lax
from jax.experimental import pallas as pl
from jax.experimental.pallas import tpu as pltpu
from jax.experimental.pallas import tpu_sc as plsc

N, E = 10000, 320000
ND, ED, NB = 128, 128, 20
HID = ND + 2 * ED
PW = 512
NPAD = 10240
RANGE = 1024
ROUNDS = 5
ACC_ROWS = RANGE + 8
DUMMY = RANGE
NC, NS, L = 2, 16, 16
EPT = E // NS
SEG = 2000
NSEG = EPT // SEG
CCAP = 2048
TRASH = SEG
K = 16
MAXCH = CCAP // K
ROWS_PT = RANGE // NS


def _mlp_body(x_ref, w1_ref, b1_ref, w2_ref, b2_ref, o_ref):
    h = jnp.dot(x_ref[...], w1_ref[...], preferred_element_type=jnp.float32)
    h = h + b1_ref[...]
    h = h * jax.nn.sigmoid(h)
    o_ref[...] = jnp.dot(h, w2_ref[...], preferred_element_type=jnp.float32) + b2_ref[...]


def _mlp(x, W1, b1, W2, b2):
    blk = 512
    return pl.pallas_call(
        _mlp_body,
        grid=(NPAD // blk,),
        in_specs=[
            pl.BlockSpec((blk, ND), lambda i: (i, 0)),
            pl.BlockSpec((ND, ND), lambda i: (0, 0)),
            pl.BlockSpec((1, ND), lambda i: (0, 0)),
            pl.BlockSpec((ND, HID), lambda i: (0, 0)),
            pl.BlockSpec((1, HID), lambda i: (0, 0)),
        ],
        out_specs=pl.BlockSpec((blk, HID), lambda i: (i, 0)),
        out_shape=jax.ShapeDtypeStruct((NPAD, HID), jnp.float32),
    )(x, W1, b1.reshape(1, ND), W2, b2.reshape(1, HID))


def _edge_pack_body(rbf_ref, env_ref, rsh_ref, wr_ref, br_ref, o_ref):
    t = jnp.dot(rbf_ref[...], wr_ref[...], preferred_element_type=jnp.float32)
    fw = (t + br_ref[...]) * env_ref[...]
    rshp = jnp.pad(rsh_ref[...], ((0, 0), (0, PW - HID - 4)))
    o_ref[...] = jnp.concatenate([fw, rshp], axis=1)


def _edge_pack(rbf, envelope, rsh4, Wr, br):
    blk = 1280
    return pl.pallas_call(
        _edge_pack_body,
        grid=(E // blk,),
        in_specs=[
            pl.BlockSpec((blk, NB), lambda i: (i, 0)),
            pl.BlockSpec((blk, 1), lambda i: (i, 0)),
            pl.BlockSpec((blk, 4), lambda i: (i, 0)),
            pl.BlockSpec((NB, HID), lambda i: (0, 0)),
            pl.BlockSpec((1, HID), lambda i: (0, 0)),
        ],
        out_specs=pl.BlockSpec((blk, PW), lambda i: (i, 0)),
        out_shape=jax.ShapeDtypeStruct((E, PW), jnp.float32),
    )(rbf, envelope, rsh4, Wr, br.reshape(1, HID))


def _sc_body(t_hbm, p_hbm, src_hbm, dst_hbm,
             x0_hbm, x1_hbm, x2_hbm, x3_hbm,
             o0_hbm, o1_hbm, o2_hbm, o3_hbm,
             acc0, acc1, acc2, acc3, ebuf, sbuf,
             dseg, sseg, posb, pst, dummy, ecomp, scomp, didx2,
             tA, tB, gPA, gPB, msgA, msgB,
             gsemA, gsemB, ssemA, ssemB):
    c = lax.axis_index("c")
    s = lax.axis_index("s")
    lanes = lax.iota(jnp.int32, L)
    sh_idx = [jnp.maximum(lanes - sh, 0) for sh in (1, 2, 4, 8)]
    sh_msk = [lanes >= sh for sh in (1, 2, 4, 8)]
    region = s * CCAP
    accs = (acc0, acc1, acc2, acc3)
    xs = (x0_hbm, x1_hbm, x2_hbm, x3_hbm)
    os_ = (o0_hbm, o1_hbm, o2_hbm, o3_hbm)
    tb = (tA, tB)
    gP = (gPA, gPB)
    msgs = (msgA, msgB)
    gsems = (gsemA, gsemB)
    ssems = (ssemA, ssemB)

    def fill(i, _):
        dummy[pl.ds(i * L, L)] = jnp.zeros((L,), jnp.int32) + (DUMMY << 10)
        return 0
    lax.fori_loop(0, CCAP // L, fill, 0)

    def per_round(r, _):
        base = (2 * r + c) * RANGE

        row0 = s * ROWS_PT
        for q in range(4):
            pltpu.sync_copy(xs[q].at[pl.ds(base + row0, ROWS_PT)],
                            accs[q].at[pl.ds(row0, ROWS_PT)])
        @pl.when(s == 0)
        def _():
            for q in range(4):
                pltpu.sync_copy(xs[q].at[pl.ds(0, 8)],
                                accs[q].at[pl.ds(RANGE, 8)])
        plsc.subcore_barrier()

        def per_seg(g, _):
            row = s * NSEG + g
            e0 = row * SEG
            pltpu.sync_copy(dst_hbm.at[row], dseg)
            pltpu.sync_copy(src_hbm.at[row], sseg)

            def scan(i, cnt):
                d = dseg[pl.ds(i * L, L)]
                dl = d - base
                m = (dl >= 0) & (dl < RANGE)
                v = jnp.where(m, 1, 0)
                for ix, mk in zip(sh_idx, sh_msk):
                    g2 = v.at[ix].get(mode="promise_in_bounds")
                    v = v + jnp.where(mk, g2, 0)
                posb[pl.ds(i * L, L)] = jnp.where(m, cnt + v - 1, TRASH) + region
                pst[pl.ds(i * L, L)] = (
                    (jnp.where(m, dl, DUMMY) << 10) + i * L + lanes)
                return cnt + v[L - 1]

            cnt = lax.fori_loop(0, SEG // L, scan, jnp.int32(0))

            pltpu.sync_copy(dummy, ebuf.at[pl.ds(region, CCAP)])
            pltpu.sync_copy(pst, ebuf.at[posb])
            pltpu.sync_copy(sseg, sbuf.at[posb])

            pltpu.sync_copy(ebuf.at[pl.ds(region, CCAP)], ecomp)
            pltpu.sync_copy(sbuf.at[pl.ds(region, CCAP)], scomp)

            def shift(i, _):
                pk = ecomp[pl.ds(i * L, L)]
                dl = jnp.minimum(jnp.maximum(pk >> 10, 0), DUMMY)
                didx2[i, pl.ds(0, L)] = dl
                ecomp[pl.ds(i * L, L)] = (pk & 1023) + e0
                sv = scomp[pl.ds(i * L, L)]
                scomp[pl.ds(i * L, L)] = jnp.minimum(jnp.maximum(sv, 0), N - 1)
                return 0
            lax.fori_loop(0, CCAP // L, shift, 0)

            nchunks = (cnt + K - 1) // K

            def step(j, _):
                for p in (0, 1):
                    q = 1 - p

                    @pl.when(j % 2 == p)
                    def _():
                        @pl.when(j < nchunks)
                        def _():
                            off = j * K
                            pltpu.async_copy(
                                t_hbm.at[scomp.at[pl.ds(off, K)]], tb[p], gsems[p])
                            pltpu.async_copy(
                                p_hbm.at[ecomp.at[pl.ds(off, K)]], gP[p], gsems[p])

                        @pl.when(j >= 1)
                        def _():
                            pltpu.make_async_copy(
                                t_hbm.at[pl.ds(0, K)], tb[q], gsems[q]).wait()
                            pltpu.make_async_copy(
                                p_hbm.at[pl.ds(0, K)], gP[q], gsems[q]).wait()

                            @pl.when(j >= 3)
                            def _():
                                pltpu.make_async_copy(
                                    x0_hbm.at[pl.ds(0, 4 * K)], msgs[q], ssems[q]).wait()

                            def edge(k, _):
                                rv = gP[q][k, pl.ds(HID, L)]
                                for grp in range(ND // L):
                                    lo = grp * L
                                    msgs[q][k, pl.ds(lo, L)] = (
                                        tb[q][k, pl.ds(lo, L)] * gP[q][k, pl.ds(lo, L)])
                                    gev = tb[q][k, pl.ds(ND + lo, L)] * gP[q][k, pl.ds(ND + lo, L)]
                                    gsv = tb[q][k, pl.ds(2 * ND + lo, L)] * gP[q][k, pl.ds(2 * ND + lo, L)]
                                    msgs[q][K + k, pl.ds(lo, L)] = (
                                        tb[q][k, pl.ds(HID + lo, L)] * gsv + gev * rv[0])
                                    msgs[q][2 * K + k, pl.ds(lo, L)] = (
                                        tb[q][k, pl.ds(HID + ED + lo, L)] * gsv + gev * rv[1])
                                    msgs[q][3 * K + k, pl.ds(lo, L)] = (
                                        tb[q][k, pl.ds(HID + 2 * ED + lo, L)] * gsv + gev * rv[2])
                                return 0

                            lax.fori_loop(0, K, edge, 0)
                            for i in range(4):
                                pltpu.async_copy(
                                    msgs[q].at[pl.ds(i * K, K)],
                                    accs[i].at[didx2.at[j - 1]], ssems[q], add=True)
                return 0

            lax.fori_loop(0, nchunks + 1, step, 0)

            for p in (0, 1):
                @pl.when(nchunks >= 1 + p)
                def _():
                    par = (nchunks - 1 - p) % 2
                    for par2 in (0, 1):
                        @pl.when(par == par2)
                        def _():
                            pltpu.make_async_copy(
                                x0_hbm.at[pl.ds(0, 4 * K)], msgs[par2],
                                ssems[par2]).wait()
            return 0

        lax.fori_loop(0, NSEG, per_seg, 0)
        plsc.subcore_barrier()

        for q in range(4):
            pltpu.sync_copy(accs[q].at[pl.ds(row0, ROWS_PT)],
                            os_[q].at[pl.ds(base + row0, ROWS_PT)])
        plsc.subcore_barrier()
        return 0

    lax.fori_loop(0, ROUNDS, per_round, 0)


def _sc_message(t, p, src, dst, xq):
    mesh = plsc.VectorSubcoreMesh(core_axis_name="c", subcore_axis_name="s",
                                  num_cores=NC, num_subcores=NS)
    f32, i32 = jnp.float32, jnp.int32
    out_t = jax.ShapeDtypeStruct((NPAD, ND), f32)
    kfn = pl.kernel(
        _sc_body,
        out_type=(out_t, out_t, out_t, out_t),
        mesh=mesh,
        scratch_types=[
            pltpu.VMEM_SHARED((ACC_ROWS, ND), f32),
            pltpu.VMEM_SHARED((ACC_ROWS, ND), f32),
            pltpu.VMEM_SHARED((ACC_ROWS, ND), f32),
            pltpu.VMEM_SHARED((ACC_ROWS, ND), f32),
            pltpu.VMEM_SHARED((NS * CCAP,), i32),
            pltpu.VMEM_SHARED((NS * CCAP,), i32),
            pltpu.VMEM((SEG,), i32),
            pltpu.VMEM((SEG,), i32),
            pltpu.VMEM((SEG,), i32),
            pltpu.VMEM((SEG,), i32),
            pltpu.VMEM((CCAP,), i32),
            pltpu.VMEM((CCAP,), i32),
            pltpu.VMEM((CCAP,), i32),
            pltpu.VMEM((MAXCH, K), i32),
            pltpu.VMEM((K, 2 * HID), f32),
            pltpu.VMEM((K, 2 * HID), f32),
            pltpu.VMEM((K, PW), f32),
            pltpu.VMEM((K, PW), f32),
            pltpu.VMEM((4 * K, ND), f32),
            pltpu.VMEM((4 * K, ND), f32),
            pltpu.SemaphoreType.DMA,
            pltpu.SemaphoreType.DMA,
            pltpu.SemaphoreType.DMA,
            pltpu.SemaphoreType.DMA,
        ],
    )
    return kfn(t, p, src, dst, *xq)


def kernel(x_scalar, x_vector, rbf, envelope, rsh, edge_index, W1, b1, W2, b2, Wr, br):
    xs_pad = jnp.pad(x_scalar, ((0, NPAD - N), (0, 0)))
    sout = _mlp(xs_pad, W1, b1, W2, b2)
    rsh4 = jnp.pad(rsh, ((0, 0), (0, 1)))
    p = _edge_pack(rbf, envelope, rsh4, Wr, br)

    xvec = x_vector.reshape(N, 3 * ED)
    t = jnp.concatenate(
        [sout, jnp.pad(xvec, ((0, NPAD - N), (0, 0)))], axis=1)
    src = edge_index[1].astype(jnp.int32).reshape(E // SEG, SEG)
    dst = edge_index[0].astype(jnp.int32).reshape(E // SEG, SEG)
    pad_n = ((0, NPAD - N), (0, 0))
    xq = [xs_pad] + [jnp.pad(x_vector[:, q, :], pad_n) for q in range(3)]

    o0, o1, o2, o3 = _sc_message(t, p, src, dst, xq)
    new_scalar = o0[:N]
    new_vector = jnp.stack([o1[:N], o2[:N], o3[:N]], axis=1)
    return new_scalar, new_vector


# speedup vs baseline: 1.2545x; 1.0004x over previous
import jax
import jax.numpy as jnp
from jax import lax
from jax.experimental import pallas as pl
from jax.experimental.pallas import tpu as pltpu
from jax.experimental.pallas import tpu_sc as plsc

N, E = 10000, 320000
ND, ED, NB = 128, 128, 20
HID = ND + 2 * ED
PW = 512
NPAD = 10240
RANGE = 1024
ROUNDS = 5
ACC_ROWS = RANGE + 8
DUMMY = RANGE
NC, NS, L = 2, 16, 16
EPT = E // NS
SEG = 2000
NSEG = EPT // SEG
CCAP = 2048
TRASH = SEG
K = 16
MAXCH = CCAP // K
ROWS_PT = RANGE // NS


def _mlp_body(x_ref, w1_ref, b1_ref, w2_ref, b2_ref, o_ref):
    h = jnp.dot(x_ref[...], w1_ref[...], preferred_element_type=jnp.float32)
    h = h + b1_ref[...]
    h = h * jax.nn.sigmoid(h)
    o_ref[...] = jnp.dot(h, w2_ref[...], preferred_element_type=jnp.float32) + b2_ref[...]


def _mlp(x, W1, b1, W2, b2):
    blk = 512
    return pl.pallas_call(
        _mlp_body,
        grid=(NPAD // blk,),
        in_specs=[
            pl.BlockSpec((blk, ND), lambda i: (i, 0)),
            pl.BlockSpec((ND, ND), lambda i: (0, 0)),
            pl.BlockSpec((1, ND), lambda i: (0, 0)),
            pl.BlockSpec((ND, HID), lambda i: (0, 0)),
            pl.BlockSpec((1, HID), lambda i: (0, 0)),
        ],
        out_specs=pl.BlockSpec((blk, HID), lambda i: (i, 0)),
        out_shape=jax.ShapeDtypeStruct((NPAD, HID), jnp.float32),
    )(x, W1, b1.reshape(1, ND), W2, b2.reshape(1, HID))


def _edge_pack_body(rbf_ref, env_ref, rsh_ref, wr_ref, br_ref, o_ref):
    t = jnp.dot(rbf_ref[...], wr_ref[...], preferred_element_type=jnp.float32)
    fw = (t + br_ref[...]) * env_ref[...]
    rshp = jnp.pad(rsh_ref[...], ((0, 0), (0, PW - HID - 4)))
    o_ref[...] = jnp.concatenate([fw, rshp], axis=1)


def _edge_pack(rbf, envelope, rsh4, Wr, br):
    blk = 1280
    return pl.pallas_call(
        _edge_pack_body,
        grid=(E // blk,),
        in_specs=[
            pl.BlockSpec((blk, NB), lambda i: (i, 0)),
            pl.BlockSpec((blk, 1), lambda i: (i, 0)),
            pl.BlockSpec((blk, 4), lambda i: (i, 0)),
            pl.BlockSpec((NB, HID), lambda i: (0, 0)),
            pl.BlockSpec((1, HID), lambda i: (0, 0)),
        ],
        out_specs=pl.BlockSpec((blk, PW), lambda i: (i, 0)),
        out_shape=jax.ShapeDtypeStruct((E, PW), jnp.float32),
    )(rbf, envelope, rsh4, Wr, br.reshape(1, HID))


def _sc_body(t_hbm, p_hbm, src_hbm, dst_hbm,
             x0_hbm, x1_hbm, x2_hbm, x3_hbm,
             o0_hbm, o1_hbm, o2_hbm, o3_hbm,
             acc0, acc1, acc2, acc3, ebuf, sbuf,
             dseg, sseg, posb, pst, dummy, ecomp, scomp, didx2,
             tA, tB, gPA, gPB, msgA, msgB,
             gsemA, gsemB, ssemA, ssemB):
    c = lax.axis_index("c")
    s = lax.axis_index("s")
    lanes = lax.iota(jnp.int32, L)
    sh_idx = [jnp.maximum(lanes - sh, 0) for sh in (1, 2, 4, 8)]
    sh_msk = [lanes >= sh for sh in (1, 2, 4, 8)]
    region = s * CCAP
    accs = (acc0, acc1, acc2, acc3)
    xs = (x0_hbm, x1_hbm, x2_hbm, x3_hbm)
    os_ = (o0_hbm, o1_hbm, o2_hbm, o3_hbm)
    tb = (tA, tB)
    gP = (gPA, gPB)
    msgs = (msgA, msgB)
    gsems = (gsemA, gsemB)
    ssems = (ssemA, ssemB)

    def fill(i, _):
        dummy[pl.ds(i * L, L)] = jnp.zeros((L,), jnp.int32) + (DUMMY << 11)
        return 0
    lax.fori_loop(0, CCAP // L, fill, 0)

    def per_round(r, _):
        base = (2 * r + c) * RANGE

        row0 = s * ROWS_PT
        for q in range(4):
            pltpu.sync_copy(xs[q].at[pl.ds(base + row0, ROWS_PT)],
                            accs[q].at[pl.ds(row0, ROWS_PT)])
        @pl.when(s == 0)
        def _():
            for q in range(4):
                pltpu.sync_copy(xs[q].at[pl.ds(0, 8)],
                                accs[q].at[pl.ds(RANGE, 8)])
        plsc.subcore_barrier()

        def per_seg(g, _):
            row = s * NSEG + g
            e0 = row * SEG
            pltpu.sync_copy(dst_hbm.at[row], dseg)
            pltpu.sync_copy(src_hbm.at[row], sseg)

            def scan(i, cnt):
                d = dseg[pl.ds(i * L, L)]
                dl = d - base
                m = (dl >= 0) & (dl < RANGE)
                v = jnp.where(m, 1, 0)
                for ix, mk in zip(sh_idx, sh_msk):
                    g2 = v.at[ix].get(mode="promise_in_bounds")
                    v = v + jnp.where(mk, g2, 0)
                posb[pl.ds(i * L, L)] = jnp.where(m, cnt + v - 1, TRASH) + region
                pst[pl.ds(i * L, L)] = (
                    (jnp.where(m, dl, DUMMY) << 11) + i * L + lanes)
                return cnt + v[L - 1]

            cnt = lax.fori_loop(0, SEG // L, scan, jnp.int32(0))

            pltpu.sync_copy(dummy, ebuf.at[pl.ds(region, CCAP)])
            pltpu.sync_copy(pst, ebuf.at[posb])
            pltpu.sync_copy(sseg, sbuf.at[posb])

            pltpu.sync_copy(ebuf.at[pl.ds(region, CCAP)], ecomp)
            pltpu.sync_copy(sbuf.at[pl.ds(region, CCAP)], scomp)

            def shift(i, _):
                pk = ecomp[pl.ds(i * L, L)]
                dl = jnp.minimum(jnp.maximum(pk >> 11, 0), DUMMY)
                didx2[i, pl.ds(0, L)] = dl
                ecomp[pl.ds(i * L, L)] = (pk & 2047) + e0
                sv = scomp[pl.ds(i * L, L)]
                scomp[pl.ds(i * L, L)] = jnp.minimum(jnp.maximum(sv, 0), N - 1)
                return 0
            lax.fori_loop(0, CCAP // L, shift, 0)

            nchunks = (cnt + K - 1) // K

            def step(j, _):
                for p in (0, 1):
                    q = 1 - p

                    @pl.when(j % 2 == p)
                    def _():
                        @pl.when(j < nchunks)
                        def _():
                            off = j * K
                            pltpu.async_copy(
                                t_hbm.at[scomp.at[pl.ds(off, K)]], tb[p], gsems[p])
                            pltpu.async_copy(
                                p_hbm.at[ecomp.at[pl.ds(off, K)]], gP[p], gsems[p])

                        @pl.when(j >= 1)
                        def _():
                            pltpu.make_async_copy(
                                t_hbm.at[pl.ds(0, K)], tb[q], gsems[q]).wait()
                            pltpu.make_async_copy(
                                p_hbm.at[pl.ds(0, K)], gP[q], gsems[q]).wait()

                            @pl.when(j >= 3)
                            def _():
                                pltpu.make_async_copy(
                                    x0_hbm.at[pl.ds(0, 4 * K)], msgs[q], ssems[q]).wait()

                            def edge(k, _):
                                rv = gP[q][k, pl.ds(HID, L)]
                                for grp in range(ND // L):
                                    lo = grp * L
                                    msgs[q][k, pl.ds(lo, L)] = (
                                        tb[q][k, pl.ds(lo, L)] * gP[q][k, pl.ds(lo, L)])
                                    gev = tb[q][k, pl.ds(ND + lo, L)] * gP[q][k, pl.ds(ND + lo, L)]
                                    gsv = tb[q][k, pl.ds(2 * ND + lo, L)] * gP[q][k, pl.ds(2 * ND + lo, L)]
                                    msgs[q][K + k, pl.ds(lo, L)] = (
                                        tb[q][k, pl.ds(HID + lo, L)] * gsv + gev * rv[0])
                                    msgs[q][2 * K + k, pl.ds(lo, L)] = (
                                        tb[q][k, pl.ds(HID + ED + lo, L)] * gsv + gev * rv[1])
                                    msgs[q][3 * K + k, pl.ds(lo, L)] = (
                                        tb[q][k, pl.ds(HID + 2 * ED + lo, L)] * gsv + gev * rv[2])
                                return 0

                            lax.fori_loop(0, K, edge, 0)
                            for i in range(4):
                                pltpu.async_copy(
                                    msgs[q].at[pl.ds(i * K, K)],
                                    accs[i].at[didx2.at[j - 1]], ssems[q], add=True)
                return 0

            lax.fori_loop(0, nchunks + 1, step, 0)

            for p in (0, 1):
                @pl.when(nchunks >= 1 + p)
                def _():
                    par = (nchunks - 1 - p) % 2
                    for par2 in (0, 1):
                        @pl.when(par == par2)
                        def _():
                            pltpu.make_async_copy(
                                x0_hbm.at[pl.ds(0, 4 * K)], msgs[par2],
                                ssems[par2]).wait()
            return 0

        lax.fori_loop(0, NSEG, per_seg, 0)
        plsc.subcore_barrier()

        for q in range(4):
            pltpu.sync_copy(accs[q].at[pl.ds(row0, ROWS_PT)],
                            os_[q].at[pl.ds(base + row0, ROWS_PT)])
        plsc.subcore_barrier()
        return 0

    lax.fori_loop(0, ROUNDS, per_round, 0)


def _sc_message(t, p, src, dst, xq):
    mesh = plsc.VectorSubcoreMesh(core_axis_name="c", subcore_axis_name="s",
                                  num_cores=NC, num_subcores=NS)
    f32, i32 = jnp.float32, jnp.int32
    out_t = jax.ShapeDtypeStruct((NPAD, ND), f32)
    kfn = pl.kernel(
        _sc_body,
        out_type=(out_t, out_t, out_t, out_t),
        mesh=mesh,
        scratch_types=[
            pltpu.VMEM_SHARED((ACC_ROWS, ND), f32),
            pltpu.VMEM_SHARED((ACC_ROWS, ND), f32),
            pltpu.VMEM_SHARED((ACC_ROWS, ND), f32),
            pltpu.VMEM_SHARED((ACC_ROWS, ND), f32),
            pltpu.VMEM_SHARED((NS * CCAP,), i32),
            pltpu.VMEM_SHARED((NS * CCAP,), i32),
            pltpu.VMEM((SEG,), i32),
            pltpu.VMEM((SEG,), i32),
            pltpu.VMEM((SEG,), i32),
            pltpu.VMEM((SEG,), i32),
            pltpu.VMEM((CCAP,), i32),
            pltpu.VMEM((CCAP,), i32),
            pltpu.VMEM((CCAP,), i32),
            pltpu.VMEM((MAXCH, K), i32),
            pltpu.VMEM((K, 2 * HID), f32),
            pltpu.VMEM((K, 2 * HID), f32),
            pltpu.VMEM((K, PW), f32),
            pltpu.VMEM((K, PW), f32),
            pltpu.VMEM((4 * K, ND), f32),
            pltpu.VMEM((4 * K, ND), f32),
            pltpu.SemaphoreType.DMA,
            pltpu.SemaphoreType.DMA,
            pltpu.SemaphoreType.DMA,
            pltpu.SemaphoreType.DMA,
        ],
    )
    return kfn(t, p, src, dst, *xq)


def kernel(x_scalar, x_vector, rbf, envelope, rsh, edge_index, W1, b1, W2, b2, Wr, br):
    xs_pad = jnp.pad(x_scalar, ((0, NPAD - N), (0, 0)))
    sout = _mlp(xs_pad, W1, b1, W2, b2)
    rsh4 = jnp.pad(rsh, ((0, 0), (0, 1)))
    p = _edge_pack(rbf, envelope, rsh4, Wr, br)

    xvec = x_vector.reshape(N, 3 * ED)
    t = jnp.concatenate(
        [sout, jnp.pad(xvec, ((0, NPAD - N), (0, 0)))], axis=1)
    src = edge_index[1].astype(jnp.int32).reshape(E // SEG, SEG)
    dst = edge_index[0].astype(jnp.int32).reshape(E // SEG, SEG)
    pad_n = ((0, NPAD - N), (0, 0))
    xq = [xs_pad] + [jnp.pad(x_vector[:, q, :], pad_n) for q in range(3)]

    o0, o1, o2, o3 = _sc_message(t, p, src, dst, xq)
    new_scalar = o0[:N]
    new_vector = jnp.stack([o1[:N], o2[:N], o3[:N]], axis=1)
    return new_scalar, new_vector
